# Initial kernel scaffold; baseline (speedup 1.0000x reference)
import jax
import jax.numpy as jnp
from jax.experimental import pallas as pl


def kernel(pred, lab):
    raise NotImplementedError("write your pallas kernel here")


# speedup vs baseline: 1.3546x; 1.3546x over previous
import functools

import jax
import jax.numpy as jnp
from jax import lax
from jax.experimental import pallas as pl
from jax.experimental.pallas import tpu as pltpu
from jax.experimental.pallas import tpu_sc as plsc

B = 128
N = 32768
TOPK_K = 5
NC, NS, L = 2, 16, 16
NW = NC * NS
ROWS_PER_W = B // NW
CHUNKS = N // L
UNROLL = 8

_mesh = plsc.VectorSubcoreMesh(core_axis_name="c", subcore_axis_name="s")


@functools.partial(
    pl.kernel,
    out_type=jax.ShapeDtypeStruct((NW, L), jnp.float32),
    mesh=_mesh,
    scratch_types=[
        pltpu.VMEM((N + L,), jnp.float32),
        pltpu.VMEM((B + L,), jnp.int32),
        pltpu.VMEM((L,), jnp.float32),
    ],
)
def _count_kernel(pred_hbm, lab_hbm, out_hbm, row_v, lab_v, cnt_v):
    wid = lax.axis_index("s") * NC + lax.axis_index("c")
    pltpu.sync_copy(lab_hbm, lab_v.at[pl.ds(0, B)])
    iota = lax.iota(jnp.int32, L)
    zero = jnp.zeros((L,), jnp.int32)
    one = jnp.ones((L,), jnp.int32)
    correct = jnp.int32(0)
    for r in range(ROWS_PER_W):
        row = wid * ROWS_PER_W + r
        pltpu.sync_copy(pred_hbm.at[row], row_v.at[pl.ds(0, N)])
        lab_scalar = lab_v[pl.ds(row, L)][0]
        lab_splat = zero + lab_scalar
        v_splat = jnp.zeros((L,), jnp.float32) + row_v[pl.ds(lab_scalar, L)][0]

        def body(c, acc):
            for j in range(UNROLL):
                base = (c * UNROLL + j) * L
                x = row_v[pl.ds(base, L)]
                idx = iota + base
                beats = (x > v_splat) | ((x == v_splat) & (idx < lab_splat))
                acc = acc + jnp.where(beats, one, zero)
            return acc

        rank_lanes = lax.fori_loop(0, CHUNKS // UNROLL, body, zero)
        rank = rank_lanes[0]
        for q in range(1, L):
            rank = rank + rank_lanes[q]
        correct = correct + jnp.where(rank < TOPK_K, 1, 0)
    cnt_v[...] = jnp.zeros((L,), jnp.float32) + correct.astype(jnp.float32)
    pltpu.sync_copy(cnt_v, out_hbm.at[wid])


@functools.partial(
    pl.kernel,
    out_type=jax.ShapeDtypeStruct((L,), jnp.float32),
    mesh=_mesh,
    scratch_types=[
        pltpu.VMEM((NW, L), jnp.float32),
        pltpu.VMEM((L,), jnp.float32),
    ],
)
def _reduce_kernel(counts_hbm, out_hbm, cnt_v, res_v):
    wid = lax.axis_index("s") * NC + lax.axis_index("c")

    @pl.when(wid == 0)
    def _():
        pltpu.sync_copy(counts_hbm, cnt_v)
        acc = jnp.zeros((L,), jnp.float32)
        for i in range(NW):
            acc = acc + cnt_v[i, :]
        res_v[...] = acc * (100.0 / B)
        pltpu.sync_copy(res_v, out_hbm)


@jax.jit
def kernel(pred, lab):
    counts = _count_kernel(pred, lab.astype(jnp.int32))
    res = _reduce_kernel(counts)
    return res[:1]


# speedup vs baseline: 2.3326x; 1.7220x over previous
import functools

import jax
import jax.numpy as jnp
from jax import lax
from jax.experimental import pallas as pl
from jax.experimental.pallas import tpu as pltpu
from jax.experimental.pallas import tpu_sc as plsc

B = 128
N = 32768
TOPK_K = 5
NC, NS, L = 2, 16, 16
NW = NC * NS
ROWS_PER_W = B // NW
CHUNKS = N // L
UNROLL = 8

_mesh = plsc.VectorSubcoreMesh(core_axis_name="c", subcore_axis_name="s")


@functools.partial(
    pl.kernel,
    out_type=jax.ShapeDtypeStruct((NW, L), jnp.float32),
    mesh=_mesh,
    scratch_types=[
        pltpu.VMEM((N + L * UNROLL,), jnp.float32),
        pltpu.VMEM((B + L,), jnp.int32),
        pltpu.VMEM((L,), jnp.float32),
    ],
)
def _count_kernel(pred_hbm, lab_hbm, out_hbm, row_v, lab_v, cnt_v):
    wid = lax.axis_index("s") * NC + lax.axis_index("c")
    pltpu.sync_copy(lab_hbm, lab_v.at[pl.ds(0, B)])
    iota = lax.iota(jnp.int32, L)
    zero = jnp.zeros((L,), jnp.int32)
    one = jnp.ones((L,), jnp.int32)
    correct = jnp.int32(0)
    for r in range(ROWS_PER_W):
        row = wid * ROWS_PER_W + r
        pltpu.sync_copy(pred_hbm.at[row], row_v.at[pl.ds(0, N)])
        lab_scalar = lab_v[pl.ds(row, L)][0]
        lab_splat = zero + lab_scalar
        v_splat = jnp.zeros((L,), jnp.float32) + row_v[pl.ds(lab_scalar, L)][0]

        def body(c, acc):
            for j in range(UNROLL):
                base = (c * UNROLL + j) * L
                x = row_v[pl.ds(base, L)]
                acc = acc + jnp.where(x > v_splat, one, zero)
            return acc

        gt_lanes = lax.fori_loop(0, CHUNKS // UNROLL, body, zero)
        gt = gt_lanes[0]
        for q in range(1, L):
            gt = gt + gt_lanes[q]

        n2 = jnp.where(gt < TOPK_K, (lab_scalar + L * UNROLL - 1) // (L * UNROLL), 0)

        def body2(c, acc):
            for j in range(UNROLL):
                base = (c * UNROLL + j) * L
                x = row_v[pl.ds(base, L)]
                idx = iota + base
                m = (x == v_splat) & (idx < lab_splat)
                acc = acc + jnp.where(m, one, zero)
            return acc

        eq_lanes = lax.fori_loop(0, n2, body2, zero)
        eq = eq_lanes[0]
        for q in range(1, L):
            eq = eq + eq_lanes[q]
        rank = gt + eq
        correct = correct + jnp.where(rank < TOPK_K, 1, 0)
    cnt_v[...] = jnp.zeros((L,), jnp.float32) + correct.astype(jnp.float32)
    pltpu.sync_copy(cnt_v, out_hbm.at[wid])


@functools.partial(
    pl.kernel,
    out_type=jax.ShapeDtypeStruct((L,), jnp.float32),
    mesh=_mesh,
    scratch_types=[
        pltpu.VMEM((NW, L), jnp.float32),
        pltpu.VMEM((L,), jnp.float32),
    ],
)
def _reduce_kernel(counts_hbm, out_hbm, cnt_v, res_v):
    wid = lax.axis_index("s") * NC + lax.axis_index("c")

    @pl.when(wid == 0)
    def _():
        pltpu.sync_copy(counts_hbm, cnt_v)
        acc = jnp.zeros((L,), jnp.float32)
        for i in range(NW):
            acc = acc + cnt_v[i, :]
        res_v[...] = acc * (100.0 / B)
        pltpu.sync_copy(res_v, out_hbm)


@jax.jit
def kernel(pred, lab):
    counts = _count_kernel(pred, lab.astype(jnp.int32))
    res = _reduce_kernel(counts)
    return res[:1]


# speedup vs baseline: 2.5250x; 1.0825x over previous
import functools

import jax
import jax.numpy as jnp
from jax import lax
from jax.experimental import pallas as pl
from jax.experimental.pallas import tpu as pltpu
from jax.experimental.pallas import tpu_sc as plsc

B = 128
N = 32768
TOPK_K = 5
NC, NS, L = 2, 16, 16
NW = NC * NS
ROWS_PER_W = B // NW
CHUNKS = N // L
UNROLL = 8

_mesh = plsc.VectorSubcoreMesh(core_axis_name="c", subcore_axis_name="s")


@functools.partial(
    pl.kernel,
    out_type=jax.ShapeDtypeStruct((NW, 128), jnp.float32),
    mesh=_mesh,
    scratch_types=[
        pltpu.VMEM((N + L * UNROLL,), jnp.float32),
        pltpu.VMEM((B + L,), jnp.int32),
        pltpu.VMEM((128,), jnp.float32),
    ],
)
def _count_kernel(pred_hbm, lab_hbm, out_hbm, row_v, lab_v, cnt_v):
    wid = lax.axis_index("s") * NC + lax.axis_index("c")
    pltpu.sync_copy(lab_hbm, lab_v.at[pl.ds(0, B)])
    iota = lax.iota(jnp.int32, L)
    zero = jnp.zeros((L,), jnp.int32)
    one = jnp.ones((L,), jnp.int32)
    correct = jnp.int32(0)
    for r in range(ROWS_PER_W):
        row = wid * ROWS_PER_W + r
        pltpu.sync_copy(pred_hbm.at[row], row_v.at[pl.ds(0, N)])
        lab_scalar = lab_v[pl.ds(row, L)][0]
        lab_splat = zero + lab_scalar
        v_splat = jnp.zeros((L,), jnp.float32) + row_v[pl.ds(lab_scalar, L)][0]

        def body(c, acc):
            for j in range(UNROLL):
                base = (c * UNROLL + j) * L
                x = row_v[pl.ds(base, L)]
                acc = acc + jnp.where(x > v_splat, one, zero)
            return acc

        gt_lanes = lax.fori_loop(0, CHUNKS // UNROLL, body, zero)
        gt = gt_lanes[0]
        for q in range(1, L):
            gt = gt + gt_lanes[q]

        n2 = jnp.where(gt < TOPK_K, (lab_scalar + L * UNROLL - 1) // (L * UNROLL), 0)

        def body2(c, acc):
            for j in range(UNROLL):
                base = (c * UNROLL + j) * L
                x = row_v[pl.ds(base, L)]
                idx = iota + base
                m = (x == v_splat) & (idx < lab_splat)
                acc = acc + jnp.where(m, one, zero)
            return acc

        eq_lanes = lax.fori_loop(0, n2, body2, zero)
        eq = eq_lanes[0]
        for q in range(1, L):
            eq = eq + eq_lanes[q]
        rank = gt + eq
        correct = correct + jnp.where(rank < TOPK_K, 1, 0)
    cnt_splat = jnp.zeros((L,), jnp.float32) + correct.astype(jnp.float32)
    for q in range(128 // L):
        cnt_v[pl.ds(q * L, L)] = cnt_splat
    pltpu.sync_copy(cnt_v, out_hbm.at[wid])


def _tc_reduce_body(counts_ref, out_ref):
    out_ref[0, 0] = jnp.sum(counts_ref[...]) * (100.0 / (B * 128.0))


_tc_reduce = pl.pallas_call(
    _tc_reduce_body,
    out_shape=jax.ShapeDtypeStruct((1, 1), jnp.float32),
    out_specs=pl.BlockSpec(memory_space=pltpu.SMEM),
)


@jax.jit
def kernel(pred, lab):
    counts = _count_kernel(pred, lab.astype(jnp.int32))
    res = _tc_reduce(counts)
    return res.reshape(1)


# speedup vs baseline: 2.7524x; 1.0901x over previous
import functools

import jax
import jax.numpy as jnp
from jax import lax
from jax.experimental import pallas as pl
from jax.experimental.pallas import tpu as pltpu
from jax.experimental.pallas import tpu_sc as plsc

B = 128
N = 32768
TOPK_K = 5
NC, NS, L = 2, 16, 16
NW = NC * NS
ROWS_PER_W = B // NW
CHUNKS = N // L
UNROLL = 8

_mesh = plsc.VectorSubcoreMesh(core_axis_name="c", subcore_axis_name="s")


@functools.partial(
    pl.kernel,
    out_type=jax.ShapeDtypeStruct((NW, 128), jnp.float32),
    mesh=_mesh,
    scratch_types=[
        pltpu.VMEM((N + L * UNROLL,), jnp.float32),
        pltpu.VMEM((N + L * UNROLL,), jnp.float32),
        pltpu.VMEM((B + L,), jnp.int32),
        pltpu.VMEM((128,), jnp.float32),
        pltpu.SemaphoreType.DMA,
        pltpu.SemaphoreType.DMA,
    ],
)
def _count_kernel(pred_hbm, lab_hbm, out_hbm, row_a, row_b, lab_v, cnt_v,
                  sem_a, sem_b):
    wid = lax.axis_index("s") * NC + lax.axis_index("c")
    pltpu.sync_copy(lab_hbm, lab_v.at[pl.ds(0, B)])
    bufs, sems, cps = [row_a, row_b], [sem_a, sem_b], [None, None]
    cps[0] = pltpu.async_copy(
        pred_hbm.at[wid * ROWS_PER_W], row_a.at[pl.ds(0, N)], sem_a)
    iota = lax.iota(jnp.int32, L)
    zero = jnp.zeros((L,), jnp.int32)
    one = jnp.ones((L,), jnp.int32)
    correct = jnp.int32(0)
    for r in range(ROWS_PER_W):
        row = wid * ROWS_PER_W + r
        if r + 1 < ROWS_PER_W:
            nxt = (r + 1) % 2
            cps[nxt] = pltpu.async_copy(
                pred_hbm.at[row + 1], bufs[nxt].at[pl.ds(0, N)], sems[nxt])
        cps[r % 2].wait()
        row_v = bufs[r % 2]
        lab_scalar = lab_v[pl.ds(row, L)][0]
        lab_splat = zero + lab_scalar
        v_splat = jnp.zeros((L,), jnp.float32) + row_v[pl.ds(lab_scalar, L)][0]

        def body(c, acc):
            for j in range(UNROLL):
                base = (c * UNROLL + j) * L
                x = row_v[pl.ds(base, L)]
                acc = acc + jnp.where(x > v_splat, one, zero)
            return acc

        gt_lanes = lax.fori_loop(0, CHUNKS // UNROLL, body, zero)
        gt = gt_lanes[0]
        for q in range(1, L):
            gt = gt + gt_lanes[q]

        n2 = jnp.where(gt < TOPK_K, (lab_scalar + L * UNROLL - 1) // (L * UNROLL), 0)

        def body2(c, acc):
            for j in range(UNROLL):
                base = (c * UNROLL + j) * L
                x = row_v[pl.ds(base, L)]
                idx = iota + base
                m = (x == v_splat) & (idx < lab_splat)
                acc = acc + jnp.where(m, one, zero)
            return acc

        eq_lanes = lax.fori_loop(0, n2, body2, zero)
        eq = eq_lanes[0]
        for q in range(1, L):
            eq = eq + eq_lanes[q]
        rank = gt + eq
        correct = correct + jnp.where(rank < TOPK_K, 1, 0)
    cnt_splat = jnp.zeros((L,), jnp.float32) + correct.astype(jnp.float32)
    for q in range(128 // L):
        cnt_v[pl.ds(q * L, L)] = cnt_splat
    pltpu.sync_copy(cnt_v, out_hbm.at[wid])


def _tc_reduce_body(counts_ref, out_ref):
    out_ref[0, 0] = jnp.sum(counts_ref[...]) * (100.0 / (B * 128.0))


_tc_reduce = pl.pallas_call(
    _tc_reduce_body,
    out_shape=jax.ShapeDtypeStruct((1, 1), jnp.float32),
    out_specs=pl.BlockSpec(memory_space=pltpu.SMEM),
)


@jax.jit
def kernel(pred, lab):
    counts = _count_kernel(pred, lab.astype(jnp.int32))
    res = _tc_reduce(counts)
    return res.reshape(1)
